# Initial kernel scaffold; baseline (speedup 1.0000x reference)
import jax
import jax.numpy as jnp
from jax.experimental import pallas as pl


def kernel(x, edge_index, W1, b1, W2, b2):
    raise NotImplementedError("write your pallas kernel here")


# speedup vs baseline: 9.3979x; 9.3979x over previous
import functools

import jax
import jax.numpy as jnp
from jax import lax
from jax.experimental import pallas as pl
from jax.experimental.pallas import tpu as pltpu
from jax.experimental.pallas import tpu_sc as plsc

N = 10000
E = 320000
IN_DIM = 128
HID_DIM = 256
OUT_DIM = 128

NC = 2
NS = 16
K = 128
EPAD = 323584
ROWS_PAD = 10112
RPT_Z = 632
RPT_O = 624
D = 128
DEG_W = 16

_mesh = plsc.VectorSubcoreMesh(core_axis_name="c", subcore_axis_name="s")


def _zero_fill(buf, nrows, width):
    def z(i, _):
        for j in range(width // 16):
            buf[i, pl.ds(j * 16, 16)] = jnp.zeros((16,), jnp.float32)
        return 0
    lax.fori_loop(0, nrows, z, 0)


def _zero_accum(zbuf, accum, s):
    r0 = s * RPT_Z
    for off, n in ((0, 128), (128, 128), (256, 128), (384, 128), (512, 120)):
        pltpu.sync_copy(zbuf.at[pl.ds(0, n)], accum.at[pl.ds(r0 + off, n)])


def _copy_out(accum, out, s):
    q0 = pl.multiple_of(s * RPT_O, 8)
    pltpu.sync_copy(accum.at[pl.ds(q0, RPT_O)], out.at[pl.ds(q0, RPT_O)])

    @pl.when(s == NS - 1)
    def _():
        tail = N - NS * RPT_O
        pltpu.sync_copy(accum.at[pl.ds(NS * RPT_O, tail)],
                        out.at[pl.ds(NS * RPT_O, tail)])


def _deg_body(dst_hbm, out0, out1, buf, idxb, accum):
    c = lax.axis_index("c")
    s = lax.axis_index("s")
    _zero_fill(buf, K, DEG_W)
    _zero_accum(buf, accum, s)

    def ones(i, _):
        buf[i, :] = jnp.ones((DEG_W,), jnp.float32)
        return 0
    lax.fori_loop(0, K, ones, 0)
    plsc.subcore_barrier()

    ept = EPAD // (NC * NS)
    base = (s * NC + c) * ept

    def chunk(i, _):
        e0 = pl.multiple_of(base + i * K, K)
        pltpu.sync_copy(dst_hbm.at[pl.ds(e0, K)], idxb.at[0])
        pltpu.sync_copy(buf, accum.at[idxb.at[0]], add=True)
        return 0
    lax.fori_loop(0, ept // K, chunk, 0)
    plsc.subcore_barrier()

    @pl.when(c == 0)
    def _():
        _copy_out(accum, out0, s)

    @pl.when(c == 1)
    def _():
        _copy_out(accum, out1, s)


_deg_kernel = functools.partial(
    pl.kernel,
    mesh=_mesh,
    out_type=[jax.ShapeDtypeStruct((N, DEG_W), jnp.float32)] * 2,
    scratch_types=[
        pltpu.VMEM((K, DEG_W), jnp.float32),
        pltpu.VMEM((2, K), jnp.int32),
        pltpu.VMEM_SHARED((ROWS_PAD, DEG_W), jnp.float32),
    ],
)(_deg_body)


def _edge_body(split_edges, src_hbm, dst_hbm, t0, t1, o0, o1,
               srcb, dstb, rows, zbuf, accum, gsem):
    c = lax.axis_index("c")
    s = lax.axis_index("s")
    _zero_fill(zbuf, K, D)
    _zero_accum(zbuf, accum, s)
    plsc.subcore_barrier()

    ept = EPAD // (NC * NS if split_edges else NS)
    base = ((s * NC + c) if split_edges else s) * ept
    nch = ept // K

    def run(tbl, out):
        def chunk(i, _):
            e0 = pl.multiple_of(base + i * K, K)
            pltpu.sync_copy(src_hbm.at[pl.ds(e0, K)], srcb.at[0])
            pltpu.sync_copy(dst_hbm.at[pl.ds(e0, K)], dstb.at[0])
            pltpu.async_copy(tbl.at[srcb.at[0]], rows.at[0], gsem).wait()
            pltpu.sync_copy(rows.at[0], accum.at[dstb.at[0]], add=True)
            return 0
        lax.fori_loop(0, nch, chunk, 0)
        plsc.subcore_barrier()
        _copy_out(accum, out, s)

    @pl.when(c == 0)
    def _():
        run(t0, o0)

    @pl.when(c == 1)
    def _():
        run(t1, o1)


def _make_edge_kernel(split_edges):
    return functools.partial(
        pl.kernel,
        mesh=_mesh,
        out_type=[jax.ShapeDtypeStruct((N, D), jnp.float32)] * 2,
        scratch_types=[
            pltpu.VMEM((2, K), jnp.int32),
            pltpu.VMEM((2, K), jnp.int32),
            pltpu.VMEM((2, K, D), jnp.float32),
            pltpu.VMEM((K, D), jnp.float32),
            pltpu.VMEM_SHARED((ROWS_PAD, D), jnp.float32),
            pltpu.SemaphoreType.DMA,
        ],
    )(functools.partial(_edge_body, split_edges))


_edge_l1 = _make_edge_kernel(False)
_edge_l2 = _make_edge_kernel(True)


BM = 1000


def _mm1_body(x_ref, d_ref, w_ref, o1_ref, o2_ref):
    xa = x_ref[...] * d_ref[...]
    h = jnp.dot(xa, w_ref[...], preferred_element_type=jnp.float32)
    o1_ref[...] = h[:, :128]
    o2_ref[...] = h[:, 128:]


def _mm2_body(alo_ref, ahi_ref, glo_ref, ghi_ref, d_ref, b1_ref, w2_ref, o_ref):
    d = d_ref[...]
    b = b1_ref[...]
    ulo = d * jnp.maximum(d * (alo_ref[...] + glo_ref[...]) + b[:, :128], 0.0)
    uhi = d * jnp.maximum(d * (ahi_ref[...] + ghi_ref[...]) + b[:, 128:], 0.0)
    w = w2_ref[...]
    o_ref[...] = (jnp.dot(ulo, w[:128, :], preferred_element_type=jnp.float32)
                  + jnp.dot(uhi, w[128:, :], preferred_element_type=jnp.float32))


def _ew_body(a0_ref, a1_ref, g_ref, d_ref, b2_ref, o_ref):
    o_ref[...] = (d_ref[...] * (a0_ref[...] + a1_ref[...] + g_ref[...])
                  + b2_ref[...])


def _row_spec(w):
    return pl.BlockSpec((BM, w), lambda i: (i, 0))


def _full_spec(h, w):
    return pl.BlockSpec((h, w), lambda i: (0, 0))


def kernel(x, edge_index, W1, b1, W2, b2):
    src = edge_index[0].astype(jnp.int32)
    dst = edge_index[1].astype(jnp.int32)
    pad = EPAD - E
    srcp = jnp.concatenate([src, jnp.zeros((pad,), jnp.int32)])
    dstp = jnp.concatenate([dst, jnp.full((pad,), N, jnp.int32)])

    c0, c1 = _deg_kernel(dstp)
    cnt = (c0 + c1)[:, 0]
    dinv = lax.rsqrt(cnt + 1.0)
    dcol = dinv[:, None]

    g1lo, g1hi = pl.pallas_call(
        _mm1_body,
        grid=(N // BM,),
        in_specs=[_row_spec(128), _row_spec(1), _full_spec(128, 256)],
        out_specs=[_row_spec(128)] * 2,
        out_shape=[jax.ShapeDtypeStruct((N, 128), jnp.float32)] * 2,
    )(x, dcol, W1)

    a1lo, a1hi = _edge_l1(srcp, dstp, g1lo, g1hi)

    g2 = pl.pallas_call(
        _mm2_body,
        grid=(N // BM,),
        in_specs=[_row_spec(128)] * 4 + [_row_spec(1), _full_spec(1, 256),
                                         _full_spec(256, 128)],
        out_specs=_row_spec(128),
        out_shape=jax.ShapeDtypeStruct((N, 128), jnp.float32),
    )(a1lo, a1hi, g1lo, g1hi, dcol, b1.reshape(1, HID_DIM), W2)

    a20, a21 = _edge_l2(srcp, dstp, g2, g2)

    out = pl.pallas_call(
        _ew_body,
        grid=(N // BM,),
        in_specs=[_row_spec(128)] * 3 + [_row_spec(1), _full_spec(1, 128)],
        out_specs=_row_spec(128),
        out_shape=jax.ShapeDtypeStruct((N, OUT_DIM), jnp.float32),
    )(a20, a21, g2, dcol, b2.reshape(1, OUT_DIM))
    return out
